# Initial kernel scaffold; baseline (speedup 1.0000x reference)
import functools

import jax
import jax.numpy as jnp
from jax import lax
from jax.experimental import pallas as pl
from jax.experimental.pallas import tpu as pltpu
from jax.experimental.pallas import tpu_sc as plsc

N = 10000
E = 320000
DP = 32
ROWS = E // 128
CHUNK_ROWS = 20
CHUNK_E = CHUNK_ROWS * 128
NCHUNKS = ROWS // CHUNK_ROWS
NWORK = 32
ROWS_PER_TILE = N // 16
B = 2560
NBLK = E // B
INV_SQRT3 = 0.5773502691896258

_mesh = plsc.VectorSubcoreMesh(core_axis_name="c", subcore_axis_name="s")


def _worker_chunks(w):
    extra = NCHUNKS % NWORK
    return jnp.where(w < extra, NCHUNKS // NWORK + 1, NCHUNKS // NWORK)


@functools.partial(
    pl.kernel,
    out_type=jax.ShapeDtypeStruct((E, DP), jnp.float32),
    mesh=_mesh,
    scratch_types=[
        pltpu.VMEM((CHUNK_ROWS, 128), jnp.int32),
        pltpu.VMEM((CHUNK_E, DP), jnp.float32),
        pltpu.SemaphoreType.DMA,
    ],
)
def _sc_gather(node_hbm, dst_hbm, x_hbm, idx_v, rows_v, sem):
    c = lax.axis_index("c")
    s = lax.axis_index("s")
    w = s * 2 + c

    def body(i, carry):
        cid = w + i * NWORK
        r0 = cid * CHUNK_ROWS
        pltpu.sync_copy(dst_hbm.at[pl.ds(r0, CHUNK_ROWS)], idx_v)
        copies = [
            pltpu.async_copy(
                node_hbm.at[idx_v.at[j]],
                rows_v.at[pl.ds(j * 128, 128)],
                sem,
            )
            for j in range(CHUNK_ROWS)
        ]
        for d in copies:
            d.wait()
        pltpu.sync_copy(rows_v, x_hbm.at[pl.ds(cid * CHUNK_E, CHUNK_E)])
        return carry

    lax.fori_loop(0, _worker_chunks(w), body, 0)


@functools.partial(
    pl.kernel,
    out_type=jax.ShapeDtypeStruct((2, N, DP), jnp.float32),
    mesh=_mesh,
    scratch_types=[
        pltpu.VMEM((CHUNK_ROWS, 128), jnp.int32),
        pltpu.VMEM((CHUNK_E, DP), jnp.float32),
        pltpu.VMEM_SHARED((N, DP), jnp.float32),
        pltpu.SemaphoreType.DMA,
    ],
)
def _sc_scatter(tp_hbm, src_hbm, zeros_hbm, out_hbm, idx_v, vals_v, acc, sem):
    c = lax.axis_index("c")
    s = lax.axis_index("s")
    w = s * 2 + c

    pltpu.sync_copy(zeros_hbm, acc.at[pl.ds(s * ROWS_PER_TILE, ROWS_PER_TILE)])
    plsc.subcore_barrier()

    def body(i, carry):
        cid = w + i * NWORK
        r0 = cid * CHUNK_ROWS
        pltpu.sync_copy(src_hbm.at[pl.ds(r0, CHUNK_ROWS)], idx_v)
        pltpu.sync_copy(tp_hbm.at[pl.ds(cid * CHUNK_E, CHUNK_E)], vals_v)
        copies = [
            pltpu.async_copy(
                vals_v.at[pl.ds(j * 128, 128)],
                acc.at[idx_v.at[j]],
                sem,
                add=True,
            )
            for j in range(CHUNK_ROWS)
        ]
        for d in copies:
            d.wait()
        return carry

    lax.fori_loop(0, _worker_chunks(w), body, 0)
    plsc.subcore_barrier()
    pltpu.sync_copy(
        acc.at[pl.ds(s * ROWS_PER_TILE, ROWS_PER_TILE)],
        out_hbm.at[c, pl.ds(s * ROWS_PER_TILE, ROWS_PER_TILE)],
    )


def _tc_body(x_ref, ea_ref, sh_ref, w1_ref, b1_ref, w2_ref, b2_ref, tp_ref):
    ea = ea_ref[...]
    h = jnp.maximum(
        jnp.dot(ea, w1_ref[...], preferred_element_type=jnp.float32) + b1_ref[...],
        0.0,
    )
    w = jnp.dot(h, w2_ref[...], preferred_element_type=jnp.float32) + b2_ref[...]
    x = x_ref[...]
    sh = sh_ref[...]
    xs = x[:, :16]
    sh_s = sh[:, 0:1]

    a0 = xs * sh_s
    out_s = a0[:, 0:1] * w[:, 0:16]
    for i in range(1, 16):
        out_s = out_s + a0[:, i : i + 1] * w[:, 16 * i : 16 * i + 16]
    for i in range(2):
        dot_i = (
            x[:, 16 + 3 * i : 17 + 3 * i] * sh[:, 1:2]
            + x[:, 17 + 3 * i : 18 + 3 * i] * sh[:, 2:3]
            + x[:, 18 + 3 * i : 19 + 3 * i] * sh[:, 3:4]
        ) * INV_SQRT3
        out_s = out_s + dot_i * w[:, 256 + 16 * i : 272 + 16 * i]

    t1 = xs[:, 0:1] * w[:, 288:292]
    for i in range(1, 16):
        t1 = t1 + xs[:, i : i + 1] * w[:, 288 + 4 * i : 292 + 4 * i]
    cols = [out_s]
    for d in range(3):
        vd = t1 * (sh[:, 1 + d : 2 + d] * INV_SQRT3)
        for i in range(2):
            vd = vd + (x[:, 16 + 3 * i + d : 17 + 3 * i + d] * sh_s) * w[
                :, 352 + 4 * i : 356 + 4 * i
            ]
        cols.append(vd)
    one = jnp.ones((x.shape[0], 1), jnp.float32)
    pad = jnp.zeros((x.shape[0], 3), jnp.float32)
    tp_ref[...] = jnp.concatenate(cols + [one, pad], axis=1)


def _tc_tp(x, ea, sh, W1, b1, W2, b2):
    return pl.pallas_call(
        _tc_body,
        grid=(NBLK,),
        in_specs=[
            pl.BlockSpec((B, DP), lambda i: (i, 0)),
            pl.BlockSpec((B, 48), lambda i: (i, 0)),
            pl.BlockSpec((B, 4), lambda i: (i, 0)),
            pl.BlockSpec((48, 48), lambda i: (0, 0)),
            pl.BlockSpec((1, 48), lambda i: (0, 0)),
            pl.BlockSpec((48, 360), lambda i: (0, 0)),
            pl.BlockSpec((1, 360), lambda i: (0, 0)),
        ],
        out_specs=pl.BlockSpec((B, DP), lambda i: (i, 0)),
        out_shape=jax.ShapeDtypeStruct((E, DP), jnp.float32),
    )(x, ea, sh, W1, b1, W2, b2)


def _div_body(p_ref, o_ref):
    tot = p_ref[0] + p_ref[1]
    cnt = jnp.maximum(tot[:, 28:29], 1.0)
    o_ref[...] = tot[:, :28] / cnt


def _finish(partials):
    return pl.pallas_call(
        _div_body,
        out_shape=jax.ShapeDtypeStruct((N, 28), jnp.float32),
    )(partials)


def kernel(node_attr, edge_index, edge_attr, edge_sh, W1, b1, W2, b2):
    edge_index = edge_index.astype(jnp.int32)
    node_pad = jnp.pad(node_attr, ((0, 0), (0, DP - node_attr.shape[1])))
    dst2d = edge_index[1].reshape(ROWS, 128)
    src2d = edge_index[0].reshape(ROWS, 128)
    x = _sc_gather(node_pad, dst2d)
    tp = _tc_tp(
        x, edge_attr, edge_sh, W1, b1.reshape(1, 48), W2, b2.reshape(1, 360)
    )
    zeros = jnp.zeros((ROWS_PER_TILE, DP), jnp.float32)
    partials = _sc_scatter(tp, src2d, zeros)
    return _finish(partials)


# speedup vs baseline: 1.0537x; 1.0537x over previous
import functools

import jax
import jax.numpy as jnp
from jax import lax
from jax.experimental import pallas as pl
from jax.experimental.pallas import tpu as pltpu
from jax.experimental.pallas import tpu_sc as plsc

N = 10000
E = 320000
DP = 32
ROWS = E // 128
CHUNK_ROWS = 20
CHUNK_E = CHUNK_ROWS * 128
NCHUNKS = ROWS // CHUNK_ROWS
NWORK = 32
NP = 10240
ROWS_PER_TILE = NP // 16
B = 1280
NBLK = E // B
INV_SQRT3 = 0.5773502691896258

def _worker_chunks(w):
    extra = NCHUNKS % NWORK
    return jnp.where(w < extra, NCHUNKS // NWORK + 1, NCHUNKS // NWORK)


@functools.lru_cache(maxsize=1)
def _sc_kernels():
    mesh = plsc.VectorSubcoreMesh(core_axis_name="c", subcore_axis_name="s")
    params = pltpu.CompilerParams(use_tc_tiling_on_sc=False)

    @functools.partial(
        pl.kernel,
        out_type=jax.ShapeDtypeStruct((E, DP), jnp.float32),
        mesh=mesh,
        compiler_params=params,
        scratch_types=[
            pltpu.VMEM((CHUNK_ROWS, 128), jnp.int32),
            pltpu.VMEM((CHUNK_E, DP), jnp.float32),
            pltpu.SemaphoreType.DMA,
        ],
    )
    def sc_gather(node_hbm, dst_hbm, x_hbm, idx_v, rows_v, sem):
        c = lax.axis_index("c")
        s = lax.axis_index("s")
        w = s * 2 + c

        def body(i, carry):
            cid = w + i * NWORK
            pltpu.sync_copy(dst_hbm.at[cid], idx_v)
            copies = [
                pltpu.async_copy(
                    node_hbm.at[idx_v.at[j]],
                    rows_v.at[pl.ds(j * 128, 128)],
                    sem,
                )
                for j in range(CHUNK_ROWS)
            ]
            for d in copies:
                d.wait()
            pltpu.sync_copy(rows_v, x_hbm.at[pl.ds(cid * CHUNK_E, CHUNK_E)])
            return carry

        lax.fori_loop(0, _worker_chunks(w), body, 0)

    @functools.partial(
        pl.kernel,
        out_type=jax.ShapeDtypeStruct((2, NP, DP), jnp.float32),
        mesh=mesh,
        compiler_params=params,
        scratch_types=[
            pltpu.VMEM((CHUNK_ROWS, 128), jnp.int32),
            pltpu.VMEM((CHUNK_E, DP), jnp.float32),
            pltpu.VMEM_SHARED((NP, DP), jnp.float32),
            pltpu.SemaphoreType.DMA,
        ],
    )
    def sc_scatter(tp_hbm, src_hbm, zeros_hbm, out_hbm, idx_v, vals_v, acc, sem):
        c = lax.axis_index("c")
        s = lax.axis_index("s")
        w = s * 2 + c

        pltpu.sync_copy(
            zeros_hbm, acc.at[pl.ds(s * ROWS_PER_TILE, ROWS_PER_TILE)]
        )
        plsc.subcore_barrier()

        def body(i, carry):
            cid = w + i * NWORK
            pltpu.sync_copy(src_hbm.at[cid], idx_v)
            pltpu.sync_copy(tp_hbm.at[pl.ds(cid * CHUNK_E, CHUNK_E)], vals_v)
            copies = [
                pltpu.async_copy(
                    vals_v.at[pl.ds(j * 128, 128)],
                    acc.at[idx_v.at[j]],
                    sem,
                    add=True,
                )
                for j in range(CHUNK_ROWS)
            ]
            for d in copies:
                d.wait()
            return carry

        lax.fori_loop(0, _worker_chunks(w), body, 0)
        plsc.subcore_barrier()
        pltpu.sync_copy(
            acc.at[pl.ds(s * ROWS_PER_TILE, ROWS_PER_TILE)],
            out_hbm.at[c, pl.ds(s * ROWS_PER_TILE, ROWS_PER_TILE)],
        )

    return sc_gather, sc_scatter


def _tc_body(x_ref, ea_ref, sh_ref, w1_ref, b1_ref, w2_ref, b2_ref, tp_ref):
    ea = ea_ref[...]
    h = jnp.maximum(
        jnp.dot(ea, w1_ref[...], preferred_element_type=jnp.float32) + b1_ref[...],
        0.0,
    )
    w = jnp.dot(h, w2_ref[...], preferred_element_type=jnp.float32) + b2_ref[...]
    x = x_ref[...]
    sh = sh_ref[...]
    xs = x[:, :16]
    sh_s = sh[:, 0:1]

    a0 = xs * sh_s
    out_s = a0[:, 0:1] * w[:, 0:16]
    for i in range(1, 16):
        out_s = out_s + a0[:, i : i + 1] * w[:, 16 * i : 16 * i + 16]
    for i in range(2):
        dot_i = (
            x[:, 16 + 3 * i : 17 + 3 * i] * sh[:, 1:2]
            + x[:, 17 + 3 * i : 18 + 3 * i] * sh[:, 2:3]
            + x[:, 18 + 3 * i : 19 + 3 * i] * sh[:, 3:4]
        ) * INV_SQRT3
        out_s = out_s + dot_i * w[:, 256 + 16 * i : 272 + 16 * i]

    t1 = xs[:, 0:1] * w[:, 288:292]
    for i in range(1, 16):
        t1 = t1 + xs[:, i : i + 1] * w[:, 288 + 4 * i : 292 + 4 * i]
    cols = [out_s]
    for d in range(3):
        vd = t1 * (sh[:, 1 + d : 2 + d] * INV_SQRT3)
        for i in range(2):
            vd = vd + (x[:, 16 + 3 * i + d : 17 + 3 * i + d] * sh_s) * w[
                :, 352 + 4 * i : 356 + 4 * i
            ]
        cols.append(vd)
    one = jnp.ones((x.shape[0], 1), jnp.float32)
    pad = jnp.zeros((x.shape[0], 3), jnp.float32)
    tp_ref[...] = jnp.concatenate(cols + [one, pad], axis=1)


def _tc_tp(x, ea, sh, W1, b1, W2, b2):
    return pl.pallas_call(
        _tc_body,
        grid=(NBLK,),
        in_specs=[
            pl.BlockSpec((B, DP), lambda i: (i, 0)),
            pl.BlockSpec((B, 48), lambda i: (i, 0)),
            pl.BlockSpec((B, 4), lambda i: (i, 0)),
            pl.BlockSpec((48, 48), lambda i: (0, 0)),
            pl.BlockSpec((1, 48), lambda i: (0, 0)),
            pl.BlockSpec((48, 360), lambda i: (0, 0)),
            pl.BlockSpec((1, 360), lambda i: (0, 0)),
        ],
        out_specs=pl.BlockSpec((B, DP), lambda i: (i, 0)),
        out_shape=jax.ShapeDtypeStruct((E, DP), jnp.float32),
    )(x, ea, sh, W1, b1, W2, b2)


def _div_body(p_ref, o_ref):
    tot = p_ref[0, :N] + p_ref[1, :N]
    cnt = jnp.maximum(tot[:, 28:29], 1.0)
    o_ref[...] = tot[:, :28] / cnt


def _finish(partials):
    return pl.pallas_call(
        _div_body,
        out_shape=jax.ShapeDtypeStruct((N, 28), jnp.float32),
    )(partials)


def kernel(node_attr, edge_index, edge_attr, edge_sh, W1, b1, W2, b2):
    edge_index = edge_index.astype(jnp.int32)
    node_pad = jnp.pad(node_attr, ((0, 0), (0, DP - node_attr.shape[1])))
    dst2d = edge_index[1].reshape(NCHUNKS, CHUNK_ROWS, 128)
    src2d = edge_index[0].reshape(NCHUNKS, CHUNK_ROWS, 128)
    sc_gather, sc_scatter = _sc_kernels()
    x = sc_gather(node_pad, dst2d)
    tp = _tc_tp(
        x, edge_attr, edge_sh, W1, b1.reshape(1, 48), W2, b2.reshape(1, 360)
    )
    zeros = jnp.zeros((ROWS_PER_TILE, DP), jnp.float32)
    partials = sc_scatter(tp, src2d, zeros)
    return _finish(partials)


# speedup vs baseline: 4.9890x; 4.7349x over previous
import functools

import jax
import jax.numpy as jnp
from jax import lax
from jax.experimental import pallas as pl
from jax.experimental.pallas import tpu as pltpu
from jax.experimental.pallas import tpu_sc as plsc

N = 10000
E = 320000
DP = 32
ROWS = E // 128
CHUNK_ROWS = 20
CHUNK_E = CHUNK_ROWS * 128
NCHUNKS = ROWS // CHUNK_ROWS
NWORK = 32
NP = 10240
ROWS_PER_TILE = NP // 16
B = 2560
NBLK = E // B
INV_SQRT3 = 0.5773502691896258

def _worker_chunks(w):
    extra = NCHUNKS % NWORK
    return jnp.where(w < extra, NCHUNKS // NWORK + 1, NCHUNKS // NWORK)


@functools.lru_cache(maxsize=1)
def _sc_kernels():
    mesh = plsc.VectorSubcoreMesh(core_axis_name="c", subcore_axis_name="s")
    params = pltpu.CompilerParams(use_tc_tiling_on_sc=False)

    @functools.partial(
        pl.kernel,
        out_type=jax.ShapeDtypeStruct((E, DP), jnp.float32),
        mesh=mesh,
        compiler_params=params,
        scratch_types=[
            pltpu.VMEM((CHUNK_ROWS, 128), jnp.int32),
            pltpu.VMEM((CHUNK_E, DP), jnp.float32),
            pltpu.SemaphoreType.DMA,
        ],
    )
    def sc_gather(node_hbm, dst_hbm, x_hbm, idx_v, rows_v, sem):
        c = lax.axis_index("c")
        s = lax.axis_index("s")
        w = s * 2 + c

        def body(i, carry):
            cid = w + i * NWORK
            pltpu.sync_copy(dst_hbm.at[cid], idx_v)
            copies = [
                pltpu.async_copy(
                    node_hbm.at[idx_v.at[j]],
                    rows_v.at[pl.ds(j * 128, 128)],
                    sem,
                )
                for j in range(CHUNK_ROWS)
            ]
            for d in copies:
                d.wait()
            pltpu.sync_copy(rows_v, x_hbm.at[pl.ds(cid * CHUNK_E, CHUNK_E)])
            return carry

        lax.fori_loop(0, _worker_chunks(w), body, 0)

    @functools.partial(
        pl.kernel,
        out_type=jax.ShapeDtypeStruct((2, NP, DP), jnp.float32),
        mesh=mesh,
        compiler_params=params,
        scratch_types=[
            pltpu.VMEM((CHUNK_ROWS, 128), jnp.int32),
            pltpu.VMEM((CHUNK_E, DP), jnp.float32),
            pltpu.VMEM_SHARED((NP, DP), jnp.float32),
            pltpu.SemaphoreType.DMA,
        ],
    )
    def sc_scatter(tp_hbm, src_hbm, zeros_hbm, out_hbm, idx_v, vals_v, acc, sem):
        c = lax.axis_index("c")
        s = lax.axis_index("s")
        w = s * 2 + c

        pltpu.sync_copy(
            zeros_hbm, acc.at[pl.ds(s * ROWS_PER_TILE, ROWS_PER_TILE)]
        )
        plsc.subcore_barrier()

        def body(i, carry):
            cid = w + i * NWORK
            pltpu.sync_copy(src_hbm.at[cid], idx_v)
            pltpu.sync_copy(tp_hbm.at[pl.ds(cid * CHUNK_E, CHUNK_E)], vals_v)
            copies = [
                pltpu.async_copy(
                    vals_v.at[pl.ds(j * 128, 128)],
                    acc.at[idx_v.at[j]],
                    sem,
                    add=True,
                )
                for j in range(CHUNK_ROWS)
            ]
            for d in copies:
                d.wait()
            return carry

        lax.fori_loop(0, _worker_chunks(w), body, 0)
        plsc.subcore_barrier()
        pltpu.sync_copy(
            acc.at[pl.ds(s * ROWS_PER_TILE, ROWS_PER_TILE)],
            out_hbm.at[c, pl.ds(s * ROWS_PER_TILE, ROWS_PER_TILE)],
        )

    return sc_gather, sc_scatter


def _np_constants():
    import numpy as np

    inv3 = INV_SQRT3
    R_A = np.zeros((DP, 288), np.float32)
    S288 = np.zeros((288, 16), np.float32)
    for i in range(16):
        for o in range(16):
            R_A[i, 16 * i + o] = 1.0
            S288[16 * i + o, o] = 1.0
    for i in range(2):
        for d in range(3):
            for o in range(16):
                R_A[16 + 3 * i + d, 256 + 16 * i + o] = inv3
    for i in range(2):
        for o in range(16):
            S288[256 + 16 * i + o, o] = 1.0
    R4 = np.zeros((DP, 64), np.float32)
    S64 = np.zeros((64, 4), np.float32)
    for i in range(16):
        for o in range(4):
            R4[i, 4 * i + o] = 1.0
            S64[4 * i + o, o] = 1.0
    G1 = np.zeros((4, DP), np.float32)
    G1[0, :16] = 1.0
    for i in range(2):
        for d in range(3):
            G1[1 + d, 16 + 3 * i + d] = 1.0
    G2 = np.zeros((4, DP), np.float32)
    for i in range(2):
        for d in range(3):
            G2[0, 16 + 3 * i + d] = 1.0
    C24 = np.zeros((DP, 24), np.float32)
    C8 = np.zeros((8, 24), np.float32)
    F24 = np.zeros((24, 12), np.float32)
    for i in range(2):
        for d in range(3):
            for o in range(4):
                C24[16 + 3 * i + d, 12 * i + 4 * d + o] = 1.0
                C8[4 * i + o, 12 * i + 4 * d + o] = 1.0
                F24[12 * i + 4 * d + o, 4 * d + o] = 1.0
    T12 = np.zeros((4, 12), np.float32)
    V12 = np.zeros((4, 12), np.float32)
    for d in range(3):
        for o in range(4):
            T12[o, 4 * d + o] = 1.0
            V12[1 + d, 4 * d + o] = inv3
    return R_A, S288, R4, S64, G1, G2, C24, C8, F24, T12, V12


_CONSTS = _np_constants()


def _tc_body(x_ref, ea_ref, sh_ref, w1_ref, b1_ref, w2_ref, b2_ref,
             ra_ref, s288_ref, r4_ref, s64_ref, g1_ref, g2_ref, c24_ref,
             c8_ref, f24_ref, t12_ref, v12_ref, tp_ref):
    f32 = jnp.float32
    ea = ea_ref[...]
    h = jnp.maximum(
        jnp.dot(ea, w1_ref[...], preferred_element_type=f32) + b1_ref[...],
        0.0,
    )
    w = jnp.dot(h, w2_ref[...], preferred_element_type=f32) + b2_ref[...]
    x = x_ref[...]
    sh = sh_ref[...]

    shx = jnp.dot(sh, g1_ref[...], preferred_element_type=f32)
    shx2 = jnp.dot(sh, g2_ref[...], preferred_element_type=f32)
    y = x * shx
    y2 = x * shx2

    a = jnp.dot(y, ra_ref[...], preferred_element_type=f32)
    out_s = jnp.dot(a * w[:, :288], s288_ref[...],
                    preferred_element_type=f32)

    xb = jnp.dot(x, r4_ref[...], preferred_element_type=f32)
    t1 = jnp.dot(xb * w[:, 384:448], s64_ref[...],
                 preferred_element_type=f32)
    y2b = jnp.dot(y2, c24_ref[...], preferred_element_type=f32)
    wb = jnp.dot(w[:, 512:520], c8_ref[...],
                 preferred_element_type=f32)
    term2 = jnp.dot(y2b * wb, f24_ref[...],
                    preferred_element_type=f32)
    t1b = jnp.dot(t1, t12_ref[...], preferred_element_type=f32)
    shb = jnp.dot(sh, v12_ref[...], preferred_element_type=f32)
    vpart = t1b * shb + term2

    one = jnp.ones((x.shape[0], 1), f32)
    pad = jnp.zeros((x.shape[0], 3), f32)
    tp_ref[...] = jnp.concatenate([out_s, vpart, one, pad], axis=1)


def _tc_tp(x, ea, sh, W1, b1, W2p, b2p):
    consts = [jnp.asarray(c) for c in _CONSTS]
    const_specs = [
        pl.BlockSpec(c.shape, lambda i: (0, 0)) for c in consts
    ]
    return pl.pallas_call(
        _tc_body,
        grid=(NBLK,),
        in_specs=[
            pl.BlockSpec((B, DP), lambda i: (i, 0)),
            pl.BlockSpec((B, 48), lambda i: (i, 0)),
            pl.BlockSpec((B, 4), lambda i: (i, 0)),
            pl.BlockSpec((48, 48), lambda i: (0, 0)),
            pl.BlockSpec((1, 48), lambda i: (0, 0)),
            pl.BlockSpec((48, 640), lambda i: (0, 0)),
            pl.BlockSpec((1, 640), lambda i: (0, 0)),
        ] + const_specs,
        out_specs=pl.BlockSpec((B, DP), lambda i: (i, 0)),
        out_shape=jax.ShapeDtypeStruct((E, DP), jnp.float32),
    )(x, ea, sh, W1, b1, W2p, b2p, *consts)


def _div_body(p_ref, o_ref):
    tot = p_ref[0, :N] + p_ref[1, :N]
    cnt = jnp.maximum(tot[:, 28:29], 1.0)
    o_ref[...] = tot[:, :28] / cnt


def _finish(partials):
    return pl.pallas_call(
        _div_body,
        out_shape=jax.ShapeDtypeStruct((N, 28), jnp.float32),
    )(partials)


def kernel(node_attr, edge_index, edge_attr, edge_sh, W1, b1, W2, b2):
    edge_index = edge_index.astype(jnp.int32)
    node_pad = jnp.pad(node_attr, ((0, 0), (0, DP - node_attr.shape[1])))
    dst2d = edge_index[1].reshape(NCHUNKS, CHUNK_ROWS, 128)
    src2d = edge_index[0].reshape(NCHUNKS, CHUNK_ROWS, 128)
    sc_gather, sc_scatter = _sc_kernels()
    x = sc_gather(node_pad, dst2d)
    W2p = jnp.zeros((48, 640), jnp.float32)
    W2p = W2p.at[:, 0:288].set(W2[:, 0:288])
    W2p = W2p.at[:, 384:448].set(W2[:, 288:352])
    W2p = W2p.at[:, 512:520].set(W2[:, 352:360])
    b2p = jnp.zeros((1, 640), jnp.float32)
    b2p = b2p.at[:, 0:288].set(b2[None, 0:288])
    b2p = b2p.at[:, 384:448].set(b2[None, 288:352])
    b2p = b2p.at[:, 512:520].set(b2[None, 352:360])
    tp = _tc_tp(x, edge_attr, edge_sh, W1, b1.reshape(1, 48), W2p, b2p)
    zeros = jnp.zeros((ROWS_PER_TILE, DP), jnp.float32)
    partials = sc_scatter(tp, src2d, zeros)
    return _finish(partials)
